# baseline (speedup 1.0000x reference)
import jax
import jax.numpy as jnp
from jax.experimental import pallas as pl
from jax.experimental.pallas import tpu as pltpu

_JACCARD_THRESH = 0.5
_NEGPOS_RATIO = 3
_NOBJ = 8
_G = 8


def _loss_kernel(tgt_ref, loc_ref, conf_ref, rows_ref, out_ref):
    b = pl.program_id(0)
    D = loc_ref.shape[3]
    C = conf_ref.shape[1]

    pf0 = rows_ref[0:1, :]
    pf1 = rows_ref[1:2, :]
    pf2 = rows_ref[2:3, :]
    pf3 = rows_ref[3:4, :]
    area_b = rows_ref[4:5, :]
    dcx = rows_ref[5:6, :]
    dcy = rows_ref[6:7, :]
    inv01w = rows_ref[7:8, :]
    inv01h = rows_ref[8:9, :]
    logw5 = rows_ref[9:10, :]
    logh5 = rows_ref[10:11, :]

    iota = jax.lax.broadcasted_iota(jnp.int32, (1, D), 1)

    def tcol(i, j):
        return tgt_ref[0, :, i * 5 + j : i * 5 + j + 1]

    ov = []
    s0c, s1c, lwc, lhc, labc = [], [], [], [], []
    for i in range(_NOBJ):
        x0, y0, x1, y1 = tcol(i, 0), tcol(i, 1), tcol(i, 2), tcol(i, 3)
        labc.append(tcol(i, 4))
        s0c.append((x0 + x1) * 0.5)
        s1c.append((y0 + y1) * 0.5)
        lwc.append(jnp.log(x1 - x0) * 5.0)
        lhc.append(jnp.log(y1 - y0) * 5.0)
        iw = jnp.maximum(jnp.minimum(x1, pf2) - jnp.maximum(x0, pf0), 0.0)
        ih = jnp.maximum(jnp.minimum(y1, pf3) - jnp.maximum(y0, pf1), 0.0)
        inter = iw * ih
        area_a = (x1 - x0) * (y1 - y0)
        ov.append(inter / (area_a + area_b - inter))

    bto = ov[0]
    bti = jnp.zeros((_G, D), jnp.int32)
    for i in range(1, _NOBJ):
        better = ov[i] > bto
        bto = jnp.where(better, ov[i], bto)
        bti = jnp.where(better, i, bti)

    big = jnp.int32(2 ** 30)
    for i in range(_NOBJ):
        m_i = jnp.max(ov[i], axis=1, keepdims=True)
        cand = jnp.where(ov[i] == m_i, iota, big)
        idx_i = jnp.min(cand, axis=1, keepdims=True)
        m = iota == idx_i
        bto = jnp.where(m, 2.0, bto)
        bti = jnp.where(m, i, bti)

    pos = bto >= _JACCARD_THRESH
    posf = pos.astype(jnp.float32)
    num_pos = jnp.sum(posf, axis=1, keepdims=True)

    s0 = jnp.zeros((_G, D), jnp.float32)
    s1 = jnp.zeros((_G, D), jnp.float32)
    lw = jnp.zeros((_G, D), jnp.float32)
    lh = jnp.zeros((_G, D), jnp.float32)
    lab = jnp.zeros((_G, D), jnp.float32)
    for i in range(_NOBJ):
        sel = bti == i
        s0 = jnp.where(sel, s0c[i], s0)
        s1 = jnp.where(sel, s1c[i], s1)
        lw = jnp.where(sel, lwc[i], lw)
        lh = jnp.where(sel, lhc[i], lh)
        lab = jnp.where(sel, labc[i], lab)

    conf_t = jnp.where(pos, lab + 1.0, 0.0).astype(jnp.int32)

    g0 = (s0 - dcx) * inv01w
    g1 = (s1 - dcy) * inv01h
    g2 = lw - logw5
    g3 = lh - logh5

    loss_l = jnp.float32(0.0)
    for g, r in ((g0, 0), (g1, 1), (g2, 2), (g3, 3)):
        d = jnp.abs(loc_ref[0, r] - g)
        sl1 = jnp.where(d < 1.0, 0.5 * d * d, d - 0.5)
        loss_l = loss_l + jnp.sum(jnp.where(pos, sl1, 0.0))

    ssum = jnp.zeros((_G, D), jnp.float32)
    picked = jnp.zeros((_G, D), jnp.float32)
    for cc in range(C):
        plane = conf_ref[0, cc]
        ssum = ssum + jnp.exp(plane)
        picked = jnp.where(conf_t == cc, plane, picked)
    loss_c = jnp.log(ssum) - picked

    masked = jnp.where(pos, 0.0, loss_c)
    bits = jax.lax.bitcast_convert_type(masked, jnp.int32)
    k = jnp.minimum(num_pos.astype(jnp.int32) * _NEGPOS_RATIO, D)

    def body(_, lohi):
        lo, hi = lohi
        mid = lo + (hi - lo + 1) // 2
        cnt = jnp.sum((bits >= mid).astype(jnp.int32), axis=1, keepdims=True)
        ok = cnt >= k
        return jnp.where(ok, mid, lo), jnp.where(ok, hi, mid - 1)

    lo0 = jnp.zeros((_G, 1), jnp.int32)
    hi0 = jnp.full((_G, 1), 0x7F7FFFFF, jnp.int32)
    lo, _ = jax.lax.fori_loop(0, 31, body, (lo0, hi0))
    vk = jax.lax.bitcast_convert_type(lo, jnp.float32)
    gt = masked > vk
    cnt_gt = jnp.sum(gt.astype(jnp.float32), axis=1, keepdims=True)
    sum_gt = jnp.sum(jnp.where(gt, masked, 0.0), axis=1, keepdims=True)
    kf = k.astype(jnp.float32)
    topk = jnp.where(k > 0, sum_gt + (kf - cnt_gt) * vk, 0.0)
    loss_c_tot = jnp.sum(jnp.where(pos, loss_c, 0.0)) + jnp.sum(topk)

    @pl.when(b == 0)
    def _init():
        out_ref[0, 0] = loss_l
        out_ref[0, 1] = loss_c_tot
        out_ref[0, 2] = jnp.sum(num_pos)

    @pl.when(b != 0)
    def _acc():
        out_ref[0, 0] += loss_l
        out_ref[0, 1] += loss_c_tot
        out_ref[0, 2] += jnp.sum(num_pos)


@jax.jit
def kernel(loc, conf, dbox_list, targets):
    B, D, C = conf.shape
    nb = B // _G
    loc_r = loc.reshape(nb, _G, D, 4).transpose(0, 3, 1, 2)
    conf_r = conf.reshape(nb, _G, D, C).transpose(0, 3, 1, 2)
    tgt_r = targets.reshape(nb, _G, _NOBJ * 5)

    dcx, dcy = dbox_list[:, 0], dbox_list[:, 1]
    dw, dh = dbox_list[:, 2], dbox_list[:, 3]
    pf0, pf1 = dcx - dw / 2.0, dcy - dh / 2.0
    pf2, pf3 = dcx + dw / 2.0, dcy + dh / 2.0
    rows = jnp.stack([
        pf0, pf1, pf2, pf3,
        (pf2 - pf0) * (pf3 - pf1),
        dcx, dcy,
        1.0 / (0.1 * dw), 1.0 / (0.1 * dh),
        jnp.log(dw) * 5.0, jnp.log(dh) * 5.0,
        jnp.zeros_like(dw),
    ])

    out = pl.pallas_call(
        _loss_kernel,
        grid=(nb,),
        in_specs=[
            pl.BlockSpec((1, _G, _NOBJ * 5), lambda b: (b, 0, 0)),
            pl.BlockSpec((1, 4, _G, D), lambda b: (b, 0, 0, 0)),
            pl.BlockSpec((1, C, _G, D), lambda b: (b, 0, 0, 0)),
            pl.BlockSpec((12, D), lambda b: (0, 0)),
        ],
        out_specs=pl.BlockSpec((1, 3), lambda b: (0, 0), memory_space=pltpu.SMEM),
        out_shape=jax.ShapeDtypeStruct((1, 3), jnp.float32),
    )(tgt_r, loc_r, conf_r, rows)

    N = out[0, 2]
    return (out[0, 0] / N, out[0, 1] / N)


# speedup vs baseline: 1.0012x; 1.0012x over previous
import jax
import jax.numpy as jnp
from jax.experimental import pallas as pl
from jax.experimental.pallas import tpu as pltpu

_JACCARD_THRESH = 0.5
_NEGPOS_RATIO = 3
_NOBJ = 8
_G = 8


def _loss_kernel(tgt_ref, loc_ref, conf_ref, rows_ref, out_ref):
    b = pl.program_id(0)
    D = loc_ref.shape[3]
    C = conf_ref.shape[1]

    pf0 = rows_ref[0:1, :]
    pf1 = rows_ref[1:2, :]
    pf2 = rows_ref[2:3, :]
    pf3 = rows_ref[3:4, :]
    area_b = rows_ref[4:5, :]
    dcx = rows_ref[5:6, :]
    dcy = rows_ref[6:7, :]
    inv01w = rows_ref[7:8, :]
    inv01h = rows_ref[8:9, :]
    logw5 = rows_ref[9:10, :]
    logh5 = rows_ref[10:11, :]

    iota = jax.lax.broadcasted_iota(jnp.int32, (1, D), 1)

    def tcol(i, j):
        return tgt_ref[0, :, i * 5 + j : i * 5 + j + 1]

    s0c, s1c, lwc, lhc, labc, idxc = [], [], [], [], [], []
    big = jnp.int32(2 ** 30)
    bto = None
    bti = jnp.zeros((_G, D), jnp.int32)
    for i in range(_NOBJ):
        x0, y0, x1, y1 = tcol(i, 0), tcol(i, 1), tcol(i, 2), tcol(i, 3)
        labc.append(tcol(i, 4))
        s0c.append((x0 + x1) * 0.5)
        s1c.append((y0 + y1) * 0.5)
        lwc.append(jnp.log(x1 - x0) * 5.0)
        lhc.append(jnp.log(y1 - y0) * 5.0)
        iw = jnp.maximum(jnp.minimum(x1, pf2) - jnp.maximum(x0, pf0), 0.0)
        ih = jnp.maximum(jnp.minimum(y1, pf3) - jnp.maximum(y0, pf1), 0.0)
        inter = iw * ih
        area_a = (x1 - x0) * (y1 - y0)
        ov_i = inter / (area_a + area_b - inter)
        m_i = jnp.max(ov_i, axis=1, keepdims=True)
        cand = jnp.where(ov_i == m_i, iota, big)
        idxc.append(jnp.min(cand, axis=1, keepdims=True))
        if i == 0:
            bto = ov_i
        else:
            better = ov_i > bto
            bto = jnp.where(better, ov_i, bto)
            bti = jnp.where(better, i, bti)

    for i in range(_NOBJ):
        m = iota == idxc[i]
        bto = jnp.where(m, 2.0, bto)
        bti = jnp.where(m, i, bti)

    pos = bto >= _JACCARD_THRESH
    posf = pos.astype(jnp.float32)
    num_pos = jnp.sum(posf, axis=1, keepdims=True)

    s0 = jnp.zeros((_G, D), jnp.float32)
    s1 = jnp.zeros((_G, D), jnp.float32)
    lw = jnp.zeros((_G, D), jnp.float32)
    lh = jnp.zeros((_G, D), jnp.float32)
    lab = jnp.zeros((_G, D), jnp.float32)
    for i in range(_NOBJ):
        sel = bti == i
        s0 = jnp.where(sel, s0c[i], s0)
        s1 = jnp.where(sel, s1c[i], s1)
        lw = jnp.where(sel, lwc[i], lw)
        lh = jnp.where(sel, lhc[i], lh)
        lab = jnp.where(sel, labc[i], lab)

    conf_t = jnp.where(pos, lab + 1.0, 0.0).astype(jnp.int32)

    g0 = (s0 - dcx) * inv01w
    g1 = (s1 - dcy) * inv01h
    g2 = lw - logw5
    g3 = lh - logh5

    loss_l = jnp.float32(0.0)
    for g, r in ((g0, 0), (g1, 1), (g2, 2), (g3, 3)):
        d = jnp.abs(loc_ref[0, r] - g)
        sl1 = jnp.where(d < 1.0, 0.5 * d * d, d - 0.5)
        loss_l = loss_l + jnp.sum(jnp.where(pos, sl1, 0.0))

    ssum = jnp.zeros((_G, D), jnp.float32)
    picked = jnp.zeros((_G, D), jnp.float32)
    for cc in range(C):
        plane = conf_ref[0, cc]
        ssum = ssum + jnp.exp(plane)
        picked = jnp.where(conf_t == cc, plane, picked)
    loss_c = jnp.log(ssum) - picked

    masked = jnp.where(pos, 0.0, loss_c)
    bits = jax.lax.bitcast_convert_type(masked, jnp.int32)
    k = jnp.minimum(num_pos.astype(jnp.int32) * _NEGPOS_RATIO, D)

    def body(_, lohi):
        lo, hi = lohi
        mid = lo + (hi - lo + 1) // 2
        cnt = jnp.sum((bits >= mid).astype(jnp.int32), axis=1, keepdims=True)
        ok = cnt >= k
        return jnp.where(ok, mid, lo), jnp.where(ok, hi, mid - 1)

    lo0 = jnp.zeros((_G, 1), jnp.int32)
    hi0 = jnp.full((_G, 1), 0x7F7FFFFF, jnp.int32)
    lo, _ = jax.lax.fori_loop(0, 31, body, (lo0, hi0))
    vk = jax.lax.bitcast_convert_type(lo, jnp.float32)
    gt = masked > vk
    cnt_gt = jnp.sum(gt.astype(jnp.float32), axis=1, keepdims=True)
    sum_gt = jnp.sum(jnp.where(gt, masked, 0.0), axis=1, keepdims=True)
    kf = k.astype(jnp.float32)
    topk = jnp.where(k > 0, sum_gt + (kf - cnt_gt) * vk, 0.0)
    loss_c_tot = jnp.sum(jnp.where(pos, loss_c, 0.0)) + jnp.sum(topk)

    @pl.when(b == 0)
    def _init():
        out_ref[0, 0] = loss_l
        out_ref[0, 1] = loss_c_tot
        out_ref[0, 2] = jnp.sum(num_pos)

    @pl.when(b != 0)
    def _acc():
        out_ref[0, 0] += loss_l
        out_ref[0, 1] += loss_c_tot
        out_ref[0, 2] += jnp.sum(num_pos)


@jax.jit
def kernel(loc, conf, dbox_list, targets):
    B, D, C = conf.shape
    nb = B // _G
    loc_r = loc.reshape(nb, _G, D, 4).transpose(0, 3, 1, 2)
    conf_r = conf.reshape(nb, _G, D, C).transpose(0, 3, 1, 2)
    tgt_r = targets.reshape(nb, _G, _NOBJ * 5)

    dcx, dcy = dbox_list[:, 0], dbox_list[:, 1]
    dw, dh = dbox_list[:, 2], dbox_list[:, 3]
    pf0, pf1 = dcx - dw / 2.0, dcy - dh / 2.0
    pf2, pf3 = dcx + dw / 2.0, dcy + dh / 2.0
    rows = jnp.stack([
        pf0, pf1, pf2, pf3,
        (pf2 - pf0) * (pf3 - pf1),
        dcx, dcy,
        1.0 / (0.1 * dw), 1.0 / (0.1 * dh),
        jnp.log(dw) * 5.0, jnp.log(dh) * 5.0,
        jnp.zeros_like(dw),
    ])

    out = pl.pallas_call(
        _loss_kernel,
        grid=(nb,),
        in_specs=[
            pl.BlockSpec((1, _G, _NOBJ * 5), lambda b: (b, 0, 0)),
            pl.BlockSpec((1, 4, _G, D), lambda b: (b, 0, 0, 0)),
            pl.BlockSpec((1, C, _G, D), lambda b: (b, 0, 0, 0)),
            pl.BlockSpec((12, D), lambda b: (0, 0)),
        ],
        out_specs=pl.BlockSpec((1, 3), lambda b: (0, 0), memory_space=pltpu.SMEM),
        out_shape=jax.ShapeDtypeStruct((1, 3), jnp.float32),
    )(tgt_r, loc_r, conf_r, rows)

    N = out[0, 2]
    return (out[0, 0] / N, out[0, 1] / N)
